# Initial kernel scaffold; baseline (speedup 1.0000x reference)
import jax
import jax.numpy as jnp
from jax.experimental import pallas as pl


def kernel(encoded, n_words, span_starts, span_ends, span_valid, span_len_table, W, b):
    raise NotImplementedError("write your pallas kernel here")


# speedup vs baseline: 25.4589x; 25.4589x over previous
import functools
import math

import jax
import jax.numpy as jnp
from jax import lax
from jax.experimental import pallas as pl
from jax.experimental.pallas import tpu as pltpu
from jax.experimental.pallas import tpu_sc as plsc

_LANES = 16
_NUM_WORKERS = 32
_IDX_PER_DMA = 128


def _table_body(nc, enc_ref, lt_ref, w1_ref, w2_ref, b_ref, out_ref):
    B, S, D = enc_ref.shape
    K = lt_ref.shape[0]
    C = _LANES
    lane = lax.broadcasted_iota(jnp.int32, (S, C), 1)
    lenb = jnp.dot(lt_ref[...].astype(jnp.bfloat16),
                   w2_ref[...].astype(jnp.bfloat16),
                   preferred_element_type=jnp.float32)
    bias = b_ref[...]

    def finish_rows(sc, lane_iota):
        scm = jnp.where(lane_iota < nc, sc, -1e30)
        rowmax = jnp.max(scm, axis=1, keepdims=True)
        amax = jnp.min(jnp.where(scm == rowmax, lane_iota, C), axis=1,
                       keepdims=True).astype(jnp.float32)
        return jnp.where(lane_iota < nc, sc,
                         jnp.where(lane_iota == nc, amax, 0.0))

    for bi in range(B):
        e = enc_ref[bi]
        m = e
        for k in range(K):
            if k > 0:
                pad = jnp.broadcast_to(e[S - 1:S, :], (k, D))
                m = jnp.maximum(m, jnp.concatenate([e[k:, :], pad], axis=0))
            sc = (jnp.dot(m.astype(jnp.bfloat16),
                          w1_ref[...].astype(jnp.bfloat16),
                          preferred_element_type=jnp.float32)
                  + lenb[k:k + 1, :]) + bias
            out_ref[pl.ds((bi * K + k) * S, S), :] = finish_rows(sc, lane)

    lane8 = lax.broadcasted_iota(jnp.int32, (8, C), 1)
    scb = jnp.broadcast_to(b_ref[...], (8, C))
    out_ref[pl.ds(B * K * S, 8), :] = finish_rows(scb, lane8)


def _make_gather_kernel(tot_rows, n_per_batch, S, K, inv_row):
    b_per_w = tot_rows // _NUM_WORKERS
    n_dma = b_per_w // _IDX_PER_DMA
    mesh = plsc.VectorSubcoreMesh(core_axis_name="c", subcore_axis_name="s")

    @functools.partial(
        pl.kernel, mesh=mesh,
        compiler_params=pltpu.CompilerParams(use_tc_tiling_on_sc=False),
        out_type=jax.ShapeDtypeStruct((tot_rows, _LANES), jnp.float32),
        scratch_types=[
            pltpu.VMEM((b_per_w,), jnp.int32),
            pltpu.VMEM((b_per_w,), jnp.int32),
            pltpu.VMEM((b_per_w,), jnp.int32),
            pltpu.VMEM((b_per_w,), jnp.int32),
            pltpu.VMEM((b_per_w, _LANES), jnp.float32),
            pltpu.SemaphoreType.DMA,
        ],
    )
    def gather_kernel(table_hbm, s_hbm, e_hbm, v_hbm, out_hbm,
                      s_v, e_v, v_v, idx_v, rows_v, sem):
        wid = lax.axis_index("s") * 2 + lax.axis_index("c")
        base = wid * b_per_w
        pltpu.sync_copy(s_hbm.at[pl.ds(base, b_per_w)], s_v)
        pltpu.sync_copy(e_hbm.at[pl.ds(base, b_per_w)], e_v)
        pltpu.sync_copy(v_hbm.at[pl.ds(base, b_per_w)], v_v)

        @pl.loop(0, b_per_w, step=_LANES)
        def _(i):
            s = s_v[pl.ds(i, _LANES)]
            e = e_v[pl.ds(i, _LANES)]
            v = v_v[pl.ds(i, _LANES)]
            bi = (base + i) // n_per_batch
            ln = e - s
            kk = jnp.maximum(jnp.minimum(ln, K), 1) - 1
            st = jnp.maximum(jnp.minimum(s, S - 1), 0)
            idx = jnp.where(v > 0, (bi * K) * S + kk * S + st, inv_row)
            idx_v[pl.ds(i, _LANES)] = idx

        copies = [
            pltpu.async_copy(
                table_hbm.at[idx_v.at[pl.ds(j * _IDX_PER_DMA, _IDX_PER_DMA)]],
                rows_v.at[pl.ds(j * _IDX_PER_DMA, _IDX_PER_DMA)], sem)
            for j in range(n_dma)
        ]
        for c in copies:
            c.wait()
        pltpu.sync_copy(rows_v, out_hbm.at[pl.ds(base, b_per_w)])

    return gather_kernel


def kernel(encoded, n_words, span_starts, span_ends, span_valid,
           span_len_table, W, b):
    del n_words
    B, S, D = encoded.shape
    K = span_len_table.shape[0]
    nc = W.shape[1]
    N = span_starts.shape[1]
    C = _LANES

    w1 = jnp.pad(W[:D, :], ((0, 0), (0, C - nc)))
    w2 = jnp.pad(W[D:, :], ((0, 0), (0, C - nc)))
    b2 = jnp.pad(b, (0, C - nc)).reshape(1, C)
    n_rows = B * K * S + 8
    table = pl.pallas_call(
        functools.partial(_table_body, nc),
        out_shape=jax.ShapeDtypeStruct((n_rows, C), jnp.float32),
    )(encoded, span_len_table, w1, w2, b2)

    align = 4096 // math.gcd(B, 4096)
    n_pad = ((N + align - 1) // align) * align
    pad = ((0, 0), (0, n_pad - N))
    s_flat = jnp.pad(span_starts, pad).reshape(-1)
    e_flat = jnp.pad(span_ends, pad).reshape(-1)
    v_flat = jnp.pad(span_valid, pad).reshape(-1).astype(jnp.int32)
    tot = B * n_pad
    gk = _make_gather_kernel(tot, n_pad, S, K, B * K * S)
    rows = gk(table, s_flat, e_flat, v_flat)

    out = rows.reshape(B, n_pad, C)
    scores = out[:, :N, :nc]
    preds = out[:, :N, nc].astype(jnp.int32)
    return scores, preds


# speedup vs baseline: 27.0607x; 1.0629x over previous
import functools
import math

import jax
import jax.numpy as jnp
from jax import lax
from jax.experimental import pallas as pl
from jax.experimental.pallas import tpu as pltpu
from jax.experimental.pallas import tpu_sc as plsc

_LANES = 16
_NUM_WORKERS = 32
_IDX_PER_DMA = 128


def _table_body(n_pad, enc_ref, lt_ref, w_ref, b_ref, s_ref, e_ref, v_ref,
                out_ref, idx_ref):
    B, S, D = enc_ref.shape
    K = lt_ref.shape[0]
    nc = w_ref.shape[1]
    N = s_ref.shape[1]
    C = _LANES
    inv_row = B * K * S

    w = w_ref[...]
    w1 = w[0:D, :].astype(jnp.bfloat16)
    w2 = w[D:, :].astype(jnp.bfloat16)
    bias = b_ref[...]
    lenb = jnp.dot(lt_ref[...].astype(jnp.bfloat16), w2,
                   preferred_element_type=jnp.float32)

    def finish_rows(sc, rows):
        lane = lax.broadcasted_iota(jnp.int32, (rows, nc), 1)
        rowmax = jnp.max(sc, axis=1, keepdims=True)
        amax = jnp.min(jnp.where(sc == rowmax, lane, nc), axis=1,
                       keepdims=True).astype(jnp.float32)
        return jnp.concatenate(
            [sc, amax, jnp.zeros((rows, C - nc - 1), jnp.float32)], axis=1)

    for bi in range(B):
        e = enc_ref[bi]
        m = e
        for k in range(K):
            if k > 0:
                pad = jnp.broadcast_to(e[S - 1:S, :], (k, D))
                m = jnp.maximum(m, jnp.concatenate([e[k:, :], pad], axis=0))
            sc = (jnp.dot(m.astype(jnp.bfloat16), w1,
                          preferred_element_type=jnp.float32)
                  + lenb[k:k + 1, :]) + bias
            out_ref[pl.ds((bi * K + k) * S, S), :] = finish_rows(sc, S)

    out_ref[pl.ds(inv_row, 8), :] = finish_rows(
        jnp.broadcast_to(bias, (8, nc)), 8)

    s = s_ref[...]
    ev = e_ref[...]
    v = v_ref[...]
    kk = jnp.maximum(jnp.minimum(ev - s, K), 1) - 1
    st = jnp.maximum(jnp.minimum(s, S - 1), 0)
    bvec = lax.broadcasted_iota(jnp.int32, (B, N), 0)
    idx = jnp.where(v, (bvec * K + kk) * S + st, inv_row)
    idx_ref[...] = jnp.full((B, n_pad), inv_row, jnp.int32)
    idx_ref[:, pl.ds(0, N)] = idx


def _make_gather_kernel(tot_rows, n_table_rows):
    b_per_w = tot_rows // _NUM_WORKERS
    n_dma = b_per_w // _IDX_PER_DMA
    mesh = plsc.VectorSubcoreMesh(core_axis_name="c", subcore_axis_name="s")

    @functools.partial(
        pl.kernel, mesh=mesh,
        compiler_params=pltpu.CompilerParams(use_tc_tiling_on_sc=False),
        out_type=jax.ShapeDtypeStruct((tot_rows, _LANES), jnp.float32),
        scratch_types=[
            pltpu.VMEM((b_per_w,), jnp.int32),
            pltpu.VMEM((b_per_w, _LANES), jnp.float32),
            pltpu.SemaphoreType.DMA,
        ],
    )
    def gather_kernel(table_hbm, i_hbm, out_hbm, idx_v, rows_v, sem):
        wid = lax.axis_index("s") * 2 + lax.axis_index("c")
        base = wid * b_per_w
        pltpu.sync_copy(i_hbm.at[pl.ds(base, b_per_w)], idx_v)
        copies = [
            pltpu.async_copy(
                table_hbm.at[idx_v.at[pl.ds(j * _IDX_PER_DMA, _IDX_PER_DMA)]],
                rows_v.at[pl.ds(j * _IDX_PER_DMA, _IDX_PER_DMA)], sem)
            for j in range(n_dma)
        ]
        for c in copies:
            c.wait()
        pltpu.sync_copy(rows_v, out_hbm.at[pl.ds(base, b_per_w)])

    return gather_kernel


def kernel(encoded, n_words, span_starts, span_ends, span_valid,
           span_len_table, W, b):
    del n_words
    B, S, D = encoded.shape
    K = span_len_table.shape[0]
    nc = W.shape[1]
    N = span_starts.shape[1]
    C = _LANES

    align = (_NUM_WORKERS * _IDX_PER_DMA) // math.gcd(B, _NUM_WORKERS * _IDX_PER_DMA)
    n_pad = ((N + align - 1) // align) * align
    n_rows = B * K * S + 8

    table, idx = pl.pallas_call(
        functools.partial(_table_body, n_pad),
        out_shape=(
            jax.ShapeDtypeStruct((n_rows, C), jnp.float32),
            jax.ShapeDtypeStruct((B, n_pad), jnp.int32),
        ),
    )(encoded, span_len_table, W, b.reshape(1, nc),
      span_starts, span_ends, span_valid)

    tot = B * n_pad
    gk = _make_gather_kernel(tot, n_rows)
    rows = gk(table, idx.reshape(-1))

    out = rows.reshape(B, n_pad, C)
    scores = out[:, :N, :nc]
    preds = out[:, :N, nc].astype(jnp.int32)
    return scores, preds


# speedup vs baseline: 30.4747x; 1.1262x over previous
import dataclasses
import functools
import math

import jax
import jax.numpy as jnp
from jax import lax
from jax.experimental import pallas as pl
from jax.experimental.pallas import tpu as pltpu
from jax.experimental.pallas import tpu_sc as plsc

_LANES = 16
_NUM_WORKERS = 32
_IDX_PER_DMA = 128


def _table_body(n_pad, enc_ref, lt_ref, w_ref, b_ref, s_ref, e_ref, v_ref,
                out_ref, idx_ref):
    B, S, D = enc_ref.shape
    K = lt_ref.shape[0]
    nc = w_ref.shape[1]
    N = s_ref.shape[1]
    C = _LANES
    inv_row = B * K * S

    w = w_ref[...]
    w1 = w[0:D, :].astype(jnp.bfloat16)
    w2 = w[D:, :].astype(jnp.bfloat16)
    bias = b_ref[...]
    lenb = jnp.dot(lt_ref[...].astype(jnp.bfloat16), w2,
                   preferred_element_type=jnp.float32)

    def finish_rows(sc, rows):
        lane = lax.broadcasted_iota(jnp.int32, (rows, nc), 1)
        rowmax = jnp.max(sc, axis=1, keepdims=True)
        amax = jnp.min(jnp.where(sc == rowmax, lane, nc), axis=1,
                       keepdims=True).astype(jnp.float32)
        return jnp.concatenate(
            [sc, amax, jnp.zeros((rows, C - nc - 1), jnp.float32)], axis=1)

    for bi in range(B):
        e = enc_ref[bi]
        m = e
        for k in range(K):
            if k > 0:
                pad = jnp.broadcast_to(e[S - 1:S, :], (k, D))
                m = jnp.maximum(m, jnp.concatenate([e[k:, :], pad], axis=0))
            sc = (jnp.dot(m.astype(jnp.bfloat16), w1,
                          preferred_element_type=jnp.float32)
                  + lenb[k:k + 1, :]) + bias
            out_ref[pl.ds((bi * K + k) * S, S), :] = finish_rows(sc, S)

    out_ref[pl.ds(inv_row, 8), :] = finish_rows(
        jnp.broadcast_to(bias, (8, nc)), 8)

    s = s_ref[...]
    ev = e_ref[...]
    v = v_ref[...]
    kk = jnp.maximum(jnp.minimum(ev - s, K), 1) - 1
    st = jnp.maximum(jnp.minimum(s, S - 1), 0)
    bvec = lax.broadcasted_iota(jnp.int32, (B, N), 0)
    idx = jnp.where(v, (bvec * K + kk) * S + st, inv_row)
    idx_ref[...] = jnp.full((B, n_pad), inv_row, jnp.int32)
    idx_ref[:, pl.ds(0, N)] = idx


def _make_gather_kernel(B, n_pad, nc):
    b_per_w = (B * n_pad) // _NUM_WORKERS
    n_dma = b_per_w // _IDX_PER_DMA
    w_per_b = _NUM_WORKERS // B
    mesh = plsc.VectorSubcoreMesh(core_axis_name="c", subcore_axis_name="s")
    cp = pltpu.CompilerParams(use_tc_tiling_on_sc=False)
    if "needs_layout_passes" in pltpu.CompilerParams.__dataclass_fields__:
        cp = dataclasses.replace(cp, needs_layout_passes=False)

    @functools.partial(
        pl.kernel, mesh=mesh,
        compiler_params=cp,
        out_type=(
            jax.ShapeDtypeStruct((B, n_pad, _LANES), jnp.float32),
            jax.ShapeDtypeStruct((B, n_pad), jnp.int32),
        ),
        scratch_types=[
            pltpu.VMEM((b_per_w,), jnp.int32),
            pltpu.VMEM((b_per_w, _LANES), jnp.float32),
            pltpu.VMEM((b_per_w,), jnp.int32),
            pltpu.SemaphoreType.DMA,
        ],
    )
    def gather_kernel(table_hbm, i_hbm, rows_hbm, preds_hbm,
                      idx_v, rows_v, preds_v, sem):
        wid = lax.axis_index("s") * 2 + lax.axis_index("c")
        bi = wid // w_per_b
        off = (wid % w_per_b) * b_per_w
        pltpu.sync_copy(i_hbm.at[bi, pl.ds(off, b_per_w)], idx_v)
        copies = [
            pltpu.async_copy(
                table_hbm.at[idx_v.at[pl.ds(j * _IDX_PER_DMA, _IDX_PER_DMA)]],
                rows_v.at[pl.ds(j * _IDX_PER_DMA, _IDX_PER_DMA)], sem)
            for j in range(n_dma)
        ]
        for c in copies:
            c.wait()

        lanes = lax.iota(jnp.int32, _LANES)
        col = jnp.full((_LANES,), nc, jnp.int32)

        @pl.loop(0, b_per_w, step=_LANES)
        def _(i):
            vals = plsc.load_gather(rows_v, [lanes + i, col])
            preds_v[pl.ds(i, _LANES)] = vals.astype(jnp.int32)

        pltpu.sync_copy(rows_v, rows_hbm.at[bi, pl.ds(off, b_per_w)])
        pltpu.sync_copy(preds_v, preds_hbm.at[bi, pl.ds(off, b_per_w)])

    return gather_kernel


def _epilogue_body(N, nc, rows_ref, pin_ref, scores_ref, preds_ref):
    B = rows_ref.shape[0]
    for bi in range(B):
        scores_ref[bi] = rows_ref[bi][0:N, 0:nc]
    preds_ref[...] = pin_ref[...][:, 0:N]


def kernel(encoded, n_words, span_starts, span_ends, span_valid,
           span_len_table, W, b):
    del n_words
    B, S, D = encoded.shape
    K = span_len_table.shape[0]
    nc = W.shape[1]
    N = span_starts.shape[1]
    C = _LANES

    align = (_NUM_WORKERS * _IDX_PER_DMA) // math.gcd(B, _NUM_WORKERS * _IDX_PER_DMA)
    n_pad = ((N + align - 1) // align) * align
    n_rows = B * K * S + 8

    table, idx = pl.pallas_call(
        functools.partial(_table_body, n_pad),
        out_shape=(
            jax.ShapeDtypeStruct((n_rows, C), jnp.float32),
            jax.ShapeDtypeStruct((B, n_pad), jnp.int32),
        ),
    )(encoded, span_len_table, W, b.reshape(1, nc),
      span_starts, span_ends, span_valid)

    rows, preds_pad = _make_gather_kernel(B, n_pad, nc)(table, idx)

    scores, preds = pl.pallas_call(
        functools.partial(_epilogue_body, N, nc),
        out_shape=(
            jax.ShapeDtypeStruct((B, N, nc), jnp.float32),
            jax.ShapeDtypeStruct((B, N), jnp.int32),
        ),
    )(rows, preds_pad)
    return scores, preds


# speedup vs baseline: 30.6379x; 1.0054x over previous
import dataclasses
import functools
import math

import jax
import jax.numpy as jnp
from jax import lax
from jax.experimental import pallas as pl
from jax.experimental.pallas import tpu as pltpu
from jax.experimental.pallas import tpu_sc as plsc

_LANES = 16
_NUM_WORKERS = 32
_IDX_PER_DMA = 128


def _table_body(n_pad, enc_ref, lt_ref, w_ref, b_ref, s_ref, e_ref, v_ref,
                out_ref, idx_ref):
    B, S, D = enc_ref.shape
    K = lt_ref.shape[0]
    nc = w_ref.shape[1]
    N = s_ref.shape[1]
    C = _LANES
    inv_row = B * K * S

    w = w_ref[...]
    w1 = w[0:D, :].astype(jnp.bfloat16)
    w2 = w[D:, :].astype(jnp.bfloat16)
    bias = b_ref[...]
    lenb = jnp.dot(lt_ref[...].astype(jnp.bfloat16), w2,
                   preferred_element_type=jnp.float32)

    def finish_rows(sc, rows):
        lane = lax.broadcasted_iota(jnp.int32, (rows, nc), 1)
        rowmax = jnp.max(sc, axis=1, keepdims=True)
        amax = jnp.min(jnp.where(sc == rowmax, lane, nc), axis=1,
                       keepdims=True).astype(jnp.float32)
        return jnp.concatenate(
            [sc, amax, jnp.zeros((rows, C - nc - 1), jnp.float32)], axis=1)

    for bi in range(B):
        e = enc_ref[bi]
        m = e
        for k in range(K):
            if k > 0:
                pad = jnp.broadcast_to(e[S - 1:S, :], (k, D))
                m = jnp.maximum(m, jnp.concatenate([e[k:, :], pad], axis=0))
            sc = (jnp.dot(m.astype(jnp.bfloat16), w1,
                          preferred_element_type=jnp.float32)
                  + lenb[k:k + 1, :]) + bias
            out_ref[pl.ds((bi * K + k) * S, S), :] = finish_rows(sc, S)

    out_ref[pl.ds(inv_row, 8), :] = finish_rows(
        jnp.broadcast_to(bias, (8, nc)), 8)

    s = s_ref[...]
    ev = e_ref[...]
    v = v_ref[...]
    kk = jnp.maximum(jnp.minimum(ev - s, K), 1) - 1
    st = jnp.maximum(jnp.minimum(s, S - 1), 0)
    bvec = lax.broadcasted_iota(jnp.int32, (B, N), 0)
    idx = jnp.where(v, (bvec * K + kk) * S + st, inv_row)
    idx_ref[...] = jnp.full((B, n_pad), inv_row, jnp.int32)
    idx_ref[:, pl.ds(0, N)] = idx


def _make_gather_kernel(B, n_pad, nc):
    b_per_w = (B * n_pad) // _NUM_WORKERS
    n_dma = b_per_w // _IDX_PER_DMA
    w_per_b = _NUM_WORKERS // B
    mesh = plsc.VectorSubcoreMesh(core_axis_name="c", subcore_axis_name="s")
    cp = pltpu.CompilerParams(use_tc_tiling_on_sc=False)
    if "needs_layout_passes" in pltpu.CompilerParams.__dataclass_fields__:
        cp = dataclasses.replace(cp, needs_layout_passes=False)

    @functools.partial(
        pl.kernel, mesh=mesh,
        compiler_params=cp,
        out_type=(
            jax.ShapeDtypeStruct((B, n_pad, _LANES), jnp.float32),
            jax.ShapeDtypeStruct((B, n_pad), jnp.int32),
        ),
        scratch_types=[
            pltpu.VMEM((b_per_w,), jnp.int32),
            pltpu.VMEM((b_per_w, _LANES), jnp.float32),
            pltpu.VMEM((b_per_w,), jnp.int32),
            pltpu.SemaphoreType.DMA,
        ],
    )
    def gather_kernel(table_hbm, i_hbm, rows_hbm, preds_hbm,
                      idx_v, rows_v, preds_v, sem):
        wid = lax.axis_index("s") * 2 + lax.axis_index("c")
        bi = wid // w_per_b
        off = (wid % w_per_b) * b_per_w
        pltpu.sync_copy(i_hbm.at[bi, pl.ds(off, b_per_w)], idx_v)
        copies = [
            pltpu.async_copy(
                table_hbm.at[idx_v.at[pl.ds(j * _IDX_PER_DMA, _IDX_PER_DMA)]],
                rows_v.at[pl.ds(j * _IDX_PER_DMA, _IDX_PER_DMA)], sem)
            for j in range(n_dma)
        ]
        for c in copies:
            c.wait()

        lanes = lax.iota(jnp.int32, _LANES)
        col = jnp.full((_LANES,), nc, jnp.int32)

        @pl.loop(0, b_per_w, step=_LANES)
        def _(i):
            vals = plsc.load_gather(rows_v, [lanes + i, col])
            preds_v[pl.ds(i, _LANES)] = vals.astype(jnp.int32)

        pltpu.sync_copy(rows_v, rows_hbm.at[bi, pl.ds(off, b_per_w)])
        pltpu.sync_copy(preds_v, preds_hbm.at[bi, pl.ds(off, b_per_w)])

    return gather_kernel


def kernel(encoded, n_words, span_starts, span_ends, span_valid,
           span_len_table, W, b):
    del n_words
    B, S, D = encoded.shape
    K = span_len_table.shape[0]
    nc = W.shape[1]
    N = span_starts.shape[1]
    C = _LANES

    align = (_NUM_WORKERS * _IDX_PER_DMA) // math.gcd(B, _NUM_WORKERS * _IDX_PER_DMA)
    n_pad = ((N + align - 1) // align) * align
    n_rows = B * K * S + 8

    table, idx = pl.pallas_call(
        functools.partial(_table_body, n_pad),
        out_shape=(
            jax.ShapeDtypeStruct((n_rows, C), jnp.float32),
            jax.ShapeDtypeStruct((B, n_pad), jnp.int32),
        ),
    )(encoded, span_len_table, W, b.reshape(1, nc),
      span_starts, span_ends, span_valid)

    rows, preds_pad = _make_gather_kernel(B, n_pad, nc)(table, idx)

    scores = lax.slice(rows, (0, 0, 0), (B, N, nc))
    preds = lax.slice(preds_pad, (0, 0), (B, N))
    return scores, preds


# speedup vs baseline: 32.8767x; 1.0731x over previous
import dataclasses
import functools
import math

import jax
import jax.numpy as jnp
from jax import lax
from jax.experimental import pallas as pl
from jax.experimental.pallas import tpu as pltpu
from jax.experimental.pallas import tpu_sc as plsc

_LANES = 16
_NUM_WORKERS = 32
_IDX_PER_DMA = 128


def _table_body(n_pad, enc_ref, lt_ref, w_ref, b_ref, s_ref, e_ref, v_ref,
                out_ref, idx_ref):
    B, S, D = enc_ref.shape
    K = lt_ref.shape[0]
    nc = w_ref.shape[1]
    N = s_ref.shape[1]
    C = _LANES
    inv_row = B * K * S

    w = w_ref[...]
    w1 = w[0:D, :].astype(jnp.bfloat16)
    w2 = w[D:, :].astype(jnp.bfloat16)
    bias = b_ref[...]
    lenb = jnp.dot(lt_ref[...].astype(jnp.bfloat16), w2,
                   preferred_element_type=jnp.float32)

    def finish_rows(sc, rows):
        lane = lax.broadcasted_iota(jnp.int32, (rows, nc), 1)
        rowmax = jnp.max(sc, axis=1, keepdims=True)
        amax = jnp.min(jnp.where(sc == rowmax, lane, nc), axis=1,
                       keepdims=True).astype(jnp.float32)
        return jnp.concatenate(
            [sc, amax, jnp.zeros((rows, C - nc - 1), jnp.float32)], axis=1)

    blocks = []
    grp = 0
    for bi in range(B):
        e = enc_ref[bi]
        m = e
        for k in range(K):
            if k > 0:
                pad = jnp.broadcast_to(e[S - 1:S, :], (k, D))
                m = jnp.maximum(m, jnp.concatenate([e[k:, :], pad], axis=0))
            sc = (jnp.dot(m.astype(jnp.bfloat16), w1,
                          preferred_element_type=jnp.float32)
                  + lenb[k:k + 1, :]) + bias
            blocks.append(finish_rows(sc, S))
            if len(blocks) == 8:
                out_ref[pl.ds(grp * S, S), :] = jnp.concatenate(blocks, axis=1)
                blocks = []
                grp += 1

    inv = finish_rows(jnp.broadcast_to(bias, (1, nc)), 1)
    out_ref[pl.ds(grp * S, 1), :] = jnp.concatenate([inv] * 8, axis=1)

    s = s_ref[...]
    ev = e_ref[...]
    v = v_ref[...]
    kk = jnp.maximum(jnp.minimum(ev - s, K), 1) - 1
    st = jnp.maximum(jnp.minimum(s, S - 1), 0)
    bvec = lax.broadcasted_iota(jnp.int32, (B, N), 0)
    bk = bvec * K + kk
    idx = jnp.where(v, ((bk // 8) * S + st) * 8 + (bk % 8), inv_row)
    idx_ref[...] = jnp.full((B, n_pad), inv_row, jnp.int32)
    idx_ref[:, pl.ds(0, N)] = idx


def _make_gather_kernel(B, n_pad, nc):
    b_per_w = (B * n_pad) // _NUM_WORKERS
    n_dma = b_per_w // _IDX_PER_DMA
    w_per_b = _NUM_WORKERS // B
    mesh = plsc.VectorSubcoreMesh(core_axis_name="c", subcore_axis_name="s")
    cp = pltpu.CompilerParams(use_tc_tiling_on_sc=False)
    if "needs_layout_passes" in pltpu.CompilerParams.__dataclass_fields__:
        cp = dataclasses.replace(cp, needs_layout_passes=False)

    @functools.partial(
        pl.kernel, mesh=mesh,
        compiler_params=cp,
        out_type=(
            jax.ShapeDtypeStruct((B, n_pad, _LANES), jnp.float32),
            jax.ShapeDtypeStruct((B, n_pad), jnp.int32),
        ),
        scratch_types=[
            pltpu.VMEM((b_per_w,), jnp.int32),
            pltpu.VMEM((b_per_w, _LANES), jnp.float32),
            pltpu.VMEM((b_per_w,), jnp.int32),
            pltpu.SemaphoreType.DMA,
        ],
    )
    def gather_kernel(table_hbm, i_hbm, rows_hbm, preds_hbm,
                      idx_v, rows_v, preds_v, sem):
        wid = lax.axis_index("s") * 2 + lax.axis_index("c")
        bi = wid // w_per_b
        off = (wid % w_per_b) * b_per_w
        pltpu.sync_copy(i_hbm.at[bi, pl.ds(off, b_per_w)], idx_v)
        copies = [
            pltpu.async_copy(
                table_hbm.at[idx_v.at[pl.ds(j * _IDX_PER_DMA, _IDX_PER_DMA)]],
                rows_v.at[pl.ds(j * _IDX_PER_DMA, _IDX_PER_DMA)], sem)
            for j in range(n_dma)
        ]
        for c in copies:
            c.wait()

        lanes = lax.iota(jnp.int32, _LANES)
        col = jnp.full((_LANES,), nc, jnp.int32)

        @pl.loop(0, b_per_w, step=_LANES)
        def _(i):
            vals = plsc.load_gather(rows_v, [lanes + i, col])
            preds_v[pl.ds(i, _LANES)] = vals.astype(jnp.int32)

        pltpu.sync_copy(rows_v, rows_hbm.at[bi, pl.ds(off, b_per_w)])
        pltpu.sync_copy(preds_v, preds_hbm.at[bi, pl.ds(off, b_per_w)])

    return gather_kernel


def kernel(encoded, n_words, span_starts, span_ends, span_valid,
           span_len_table, W, b):
    del n_words
    B, S, D = encoded.shape
    K = span_len_table.shape[0]
    nc = W.shape[1]
    N = span_starts.shape[1]
    C = _LANES

    align = (_NUM_WORKERS * _IDX_PER_DMA) // math.gcd(B, _NUM_WORKERS * _IDX_PER_DMA)
    n_pad = ((N + align - 1) // align) * align
    n_rows = B * K * S + 8

    table_p, idx = pl.pallas_call(
        functools.partial(_table_body, n_pad),
        out_shape=(
            jax.ShapeDtypeStruct((n_rows // 8, 8 * C), jnp.float32),
            jax.ShapeDtypeStruct((B, n_pad), jnp.int32),
        ),
    )(encoded, span_len_table, W, b.reshape(1, nc),
      span_starts, span_ends, span_valid)

    table = table_p.reshape(n_rows, C)

    rows, preds_pad = _make_gather_kernel(B, n_pad, nc)(table, idx)

    scores = lax.slice(rows, (0, 0, 0), (B, N, nc))
    preds = lax.slice(preds_pad, (0, 0), (B, N))
    return scores, preds
